# baseline (speedup 1.0000x reference)
import functools

import jax
import jax.numpy as jnp
from jax import lax
from jax.experimental import pallas as pl
from jax.experimental.pallas import tpu as pltpu
from jax.experimental.pallas import tpu_sc as plsc

N_CLASSES = 19
EPS = 1e-08
IGNORE_INDEX = 0

_HB = 256
_NBINS = 384
_NW = 32
_L = 16


def _label_kernel(out_ref, tgt_ref, comb_ref):
    x = out_ref[0]
    lt = tgt_ref[0]

    best_val = x[0]
    best_idx = jnp.zeros_like(lt)
    for c in range(1, N_CLASSES):
        cur = x[c]
        gt = cur > best_val
        best_idx = jnp.where(gt, jnp.int32(c), best_idx)
        best_val = jnp.where(gt, cur, best_val)

    valid = jnp.logical_and(lt != IGNORE_INDEX,
                            jnp.logical_and(lt >= 0, lt < N_CLASSES))
    comb = jnp.where(valid, lt * N_CLASSES + best_idx,
                     jnp.int32(N_CLASSES * N_CLASSES))
    comb_ref[...] = comb.reshape(comb_ref.shape)


def _make_hist_call(n_pix):
    n_per_w = n_pix // _NW
    rows_per_w = n_per_w // 128

    nsub = 4
    sub_rows = rows_per_w // nsub

    @functools.partial(
        pl.kernel,
        mesh=plsc.VectorSubcoreMesh(core_axis_name="c", subcore_axis_name="s"),
        out_type=jax.ShapeDtypeStruct((_NW * _NBINS // 128, 128), jnp.float32),
        scratch_types=[
            pltpu.VMEM((rows_per_w, 128), jnp.int32),
            pltpu.VMEM((_L, _NBINS), jnp.float32),
            pltpu.VMEM((_NBINS // 128, 128), jnp.float32),
        ] + [pltpu.SemaphoreType.DMA] * nsub,
        compiler_params=pltpu.CompilerParams(
            needs_layout_passes=False, use_tc_tiling_on_sc=False),
    )
    def hist_call(comb_hbm, hist_hbm, idx_v, hist16_v, hist1_v, *sems):
        wid = lax.axis_index("s") * 2 + lax.axis_index("c")
        base = wid * rows_per_w
        copies = [
            pltpu.async_copy(
                comb_hbm.at[pl.ds(base + s * sub_rows, sub_rows), :],
                idx_v.at[pl.ds(s * sub_rows, sub_rows), :], sems[s])
            for s in range(nsub)
        ]

        zeros16 = jnp.zeros((_L,), jnp.float32)
        for r in range(_L):
            for cb in range(_NBINS // _L):
                hist16_v[r, pl.ds(cb * _L, _L)] = zeros16

        lane = lax.iota(jnp.int32, _L)
        ones16 = jnp.ones((_L,), jnp.float32)

        for s in range(nsub):
            copies[s].wait()

            @plsc.parallel_loop(s * sub_rows, (s + 1) * sub_rows, 1, unroll=2)
            def _(r):
                for j in range(128 // _L):
                    idxv = idx_v[r, pl.ds(j * _L, _L)]
                    plsc.addupdate_scatter(hist16_v, [lane, idxv], ones16)

        for cb in range(_NBINS // _L):
            acc = hist16_v[0, pl.ds(cb * _L, _L)]
            for r in range(1, _L):
                acc = acc + hist16_v[r, pl.ds(cb * _L, _L)]
            hist1_v[(cb * _L) // 128, pl.ds((cb * _L) % 128, _L)] = acc
        pltpu.sync_copy(
            hist1_v,
            hist_hbm.at[pl.ds(wid * (_NBINS // 128), _NBINS // 128), :])

    return hist_call


def _dice_kernel(*refs):
    h_refs, dsc_ref = refs[:-1], refs[-1]
    hmat = h_refs[0][...]
    for r in h_refs[1:]:
        hmat = hmat + r[...]
    nrows = _NW * (_NBINS // 128)
    rr = jax.lax.broadcasted_iota(jnp.int32, (1, nrows), 1)
    i = jax.lax.broadcasted_iota(jnp.int32, (N_CLASSES, 128), 0)
    c = jax.lax.broadcasted_iota(jnp.int32, (N_CLASSES, 128), 1)
    one = jnp.float32(1.0)
    zero = jnp.float32(0.0)
    dims = (((1,), (1,)), ((), ()))
    tp = jnp.zeros((N_CLASSES, 1), jnp.float32)
    fp = tp
    fn = tp
    for g in range(_NBINS // 128):
        m_g = jnp.where(rr % (_NBINS // 128) == g, one, zero)
        hsum_g = lax.dot_general(m_g, hmat, (((1,), (0,)), ((), ())),
                                 preferred_element_type=jnp.float32)
        k = g * 128 + c
        live = k < N_CLASSES * N_CLASSES
        s_tp = jnp.where(jnp.logical_and(k == (N_CLASSES + 1) * i, live),
                         one, zero)
        s_fp = jnp.where(jnp.logical_and(k // N_CLASSES == i, live), one, zero)
        s_fn = jnp.where(jnp.logical_and(k % N_CLASSES == i, live), one, zero)
        tp = tp + lax.dot_general(s_tp, hsum_g, dims,
                                  preferred_element_type=jnp.float32)
        fp = fp + lax.dot_general(s_fp, hsum_g, dims,
                                  preferred_element_type=jnp.float32)
        fn = fn + lax.dot_general(s_fn, hsum_g, dims,
                                  preferred_element_type=jnp.float32)
    precision = tp / (fp + EPS)
    recall = tp / (fn + EPS)
    dsc = 2.0 * precision * recall / (precision + recall + EPS)
    gt_empty = (tp + fn) == 0
    pred_empty = (tp + fp) == 0
    nan = jnp.float32(jnp.nan)
    dsc = jnp.where(jnp.logical_and(gt_empty, pred_empty), nan, dsc)
    dsc = jnp.where(jnp.logical_and(gt_empty, ~pred_empty), zero, dsc)
    row = jax.lax.broadcasted_iota(jnp.int32, (N_CLASSES, 1), 0)
    dsc = jnp.where(row == IGNORE_INDEX, nan, dsc)
    dsc_ref[...] = dsc


_NCHUNKS = 2


@jax.jit
def kernel(output, target):
    bsz, nc, hh, ww = output.shape
    target = target.astype(jnp.int32)
    nh = hh // _HB
    rows_per_blk = _HB * ww // 128
    nb = bsz // _NCHUNKS
    hist_call = _make_hist_call(nb * hh * ww)
    hists = []
    for chunk in range(_NCHUNKS):
        b0 = chunk * nb
        comb = pl.pallas_call(
            _label_kernel,
            grid=(nb, nh),
            in_specs=[
                pl.BlockSpec((1, nc, _HB, ww),
                             lambda b, h, b0=b0: (b0 + b, 0, h, 0)),
                pl.BlockSpec((1, _HB, ww), lambda b, h, b0=b0: (b0 + b, h, 0)),
            ],
            out_specs=pl.BlockSpec((rows_per_blk, 128),
                                   lambda b, h: (b * nh + h, 0)),
            out_shape=jax.ShapeDtypeStruct((nb * hh * ww // 128, 128),
                                           jnp.int32),
        )(output, target)
        hists.append(hist_call(comb))
    dsc = pl.pallas_call(
        _dice_kernel,
        out_shape=jax.ShapeDtypeStruct((N_CLASSES, 1), jnp.float32),
    )(*hists)
    return dsc.reshape(N_CLASSES)


# speedup vs baseline: 1.0239x; 1.0239x over previous
import functools

import jax
import jax.numpy as jnp
from jax import lax
from jax.experimental import pallas as pl
from jax.experimental.pallas import tpu as pltpu
from jax.experimental.pallas import tpu_sc as plsc

N_CLASSES = 19
EPS = 1e-08
IGNORE_INDEX = 0

_HB = 256
_NBINS = 384
_NW = 32
_L = 16


def _label_kernel(out_ref, tgt_ref, comb_ref):
    x = out_ref[0]
    lt = tgt_ref[0]

    best_val = x[0]
    best_idx = jnp.zeros_like(lt)
    for c in range(1, N_CLASSES):
        cur = x[c]
        gt = cur > best_val
        best_idx = jnp.where(gt, jnp.int32(c), best_idx)
        best_val = jnp.where(gt, cur, best_val)

    valid = jnp.logical_and(lt != IGNORE_INDEX,
                            jnp.logical_and(lt >= 0, lt < N_CLASSES))
    comb = jnp.where(valid, lt * N_CLASSES + best_idx,
                     jnp.int32(N_CLASSES * N_CLASSES))
    comb_ref[...] = comb.reshape(comb_ref.shape)


def _make_hist_call(n_pix):
    n_per_w = n_pix // _NW
    rows_per_w = n_per_w // 128

    @functools.partial(
        pl.kernel,
        mesh=plsc.VectorSubcoreMesh(core_axis_name="c", subcore_axis_name="s"),
        out_type=jax.ShapeDtypeStruct((_NW * _NBINS // 128, 128), jnp.float32),
        scratch_types=[
            pltpu.VMEM((rows_per_w, 128), jnp.int32),
            pltpu.VMEM((_L, _NBINS), jnp.float32),
            pltpu.VMEM((_NBINS // 128, 128), jnp.float32),
        ],
        compiler_params=pltpu.CompilerParams(
            needs_layout_passes=False, use_tc_tiling_on_sc=False),
    )
    def hist_call(comb_hbm, hist_hbm, idx_v, hist16_v, hist1_v):
        wid = lax.axis_index("s") * 2 + lax.axis_index("c")
        pltpu.sync_copy(comb_hbm.at[pl.ds(wid * rows_per_w, rows_per_w), :],
                        idx_v)

        zeros16 = jnp.zeros((_L,), jnp.float32)
        for r in range(_L):
            for cb in range(_NBINS // _L):
                hist16_v[r, pl.ds(cb * _L, _L)] = zeros16

        lane = lax.iota(jnp.int32, _L)
        ones16 = jnp.ones((_L,), jnp.float32)

        @plsc.parallel_loop(0, rows_per_w, 1, unroll=2)
        def _(r):
            for j in range(128 // _L):
                idxv = idx_v[r, pl.ds(j * _L, _L)]
                plsc.addupdate_scatter(hist16_v, [lane, idxv], ones16)

        for cb in range(_NBINS // _L):
            acc = hist16_v[0, pl.ds(cb * _L, _L)]
            for r in range(1, _L):
                acc = acc + hist16_v[r, pl.ds(cb * _L, _L)]
            hist1_v[(cb * _L) // 128, pl.ds((cb * _L) % 128, _L)] = acc
        pltpu.sync_copy(
            hist1_v,
            hist_hbm.at[pl.ds(wid * (_NBINS // 128), _NBINS // 128), :])

    return hist_call


def _dice_kernel(*refs):
    h_refs, dsc_ref = refs[:-1], refs[-1]
    hmat = h_refs[0][...]
    for r in h_refs[1:]:
        hmat = hmat + r[...]
    nrows = _NW * (_NBINS // 128)
    rr = jax.lax.broadcasted_iota(jnp.int32, (1, nrows), 1)
    i = jax.lax.broadcasted_iota(jnp.int32, (N_CLASSES, 128), 0)
    c = jax.lax.broadcasted_iota(jnp.int32, (N_CLASSES, 128), 1)
    one = jnp.float32(1.0)
    zero = jnp.float32(0.0)
    dims = (((1,), (1,)), ((), ()))
    tp = jnp.zeros((N_CLASSES, 1), jnp.float32)
    fp = tp
    fn = tp
    for g in range(_NBINS // 128):
        m_g = jnp.where(rr % (_NBINS // 128) == g, one, zero)
        hsum_g = lax.dot_general(m_g, hmat, (((1,), (0,)), ((), ())),
                                 preferred_element_type=jnp.float32)
        k = g * 128 + c
        live = k < N_CLASSES * N_CLASSES
        s_tp = jnp.where(jnp.logical_and(k == (N_CLASSES + 1) * i, live),
                         one, zero)
        s_fp = jnp.where(jnp.logical_and(k // N_CLASSES == i, live), one, zero)
        s_fn = jnp.where(jnp.logical_and(k % N_CLASSES == i, live), one, zero)
        tp = tp + lax.dot_general(s_tp, hsum_g, dims,
                                  preferred_element_type=jnp.float32)
        fp = fp + lax.dot_general(s_fp, hsum_g, dims,
                                  preferred_element_type=jnp.float32)
        fn = fn + lax.dot_general(s_fn, hsum_g, dims,
                                  preferred_element_type=jnp.float32)
    precision = tp / (fp + EPS)
    recall = tp / (fn + EPS)
    dsc = 2.0 * precision * recall / (precision + recall + EPS)
    gt_empty = (tp + fn) == 0
    pred_empty = (tp + fp) == 0
    nan = jnp.float32(jnp.nan)
    dsc = jnp.where(jnp.logical_and(gt_empty, pred_empty), nan, dsc)
    dsc = jnp.where(jnp.logical_and(gt_empty, ~pred_empty), zero, dsc)
    row = jax.lax.broadcasted_iota(jnp.int32, (N_CLASSES, 1), 0)
    dsc = jnp.where(row == IGNORE_INDEX, nan, dsc)
    dsc_ref[...] = dsc


_CHUNKS = (6, 2)


@jax.jit
def kernel(output, target):
    bsz, nc, hh, ww = output.shape
    target = target.astype(jnp.int32)
    nh = hh // _HB
    rows_per_blk = _HB * ww // 128
    hists = []
    b0 = 0
    for nb in _CHUNKS:
        hist_call = _make_hist_call(nb * hh * ww)
        comb = pl.pallas_call(
            _label_kernel,
            grid=(nb, nh),
            in_specs=[
                pl.BlockSpec((1, nc, _HB, ww),
                             lambda b, h, b0=b0: (b0 + b, 0, h, 0)),
                pl.BlockSpec((1, _HB, ww), lambda b, h, b0=b0: (b0 + b, h, 0)),
            ],
            out_specs=pl.BlockSpec((rows_per_blk, 128),
                                   lambda b, h: (b * nh + h, 0)),
            out_shape=jax.ShapeDtypeStruct((nb * hh * ww // 128, 128),
                                           jnp.int32),
        )(output, target)
        hists.append(hist_call(comb))
        b0 += nb
    dsc = pl.pallas_call(
        _dice_kernel,
        out_shape=jax.ShapeDtypeStruct((N_CLASSES, 1), jnp.float32),
    )(*hists)
    return dsc.reshape(N_CLASSES)
